# baseline (speedup 1.0000x reference)
import functools

import jax
import jax.numpy as jnp
from jax import lax
from jax.experimental import pallas as pl
from jax.experimental.pallas import tpu as pltpu
from jax.experimental.pallas import tpu_sc as plsc

D = 512
NC = 2
NS = 16
NW = NC * NS
ROWS = D // NW

_mesh = plsc.VectorSubcoreMesh(core_axis_name="c", subcore_axis_name="s")


@functools.partial(
    pl.kernel,
    mesh=_mesh,
    out_type=(
        jax.ShapeDtypeStruct((D, D), jnp.int32),
        jax.ShapeDtypeStruct((16,), jnp.float32),
    ),
    scratch_types=[
        pltpu.VMEM((ROWS, D), jnp.int32),
        pltpu.VMEM((16,), jnp.int32),
        pltpu.VMEM((16,), jnp.float32),
    ],
    compiler_params=pltpu.CompilerParams(needs_layout_passes=False),
)
def _graphgen_sc(ev_hbm, mat_hbm, out_hbm, aux_hbm, slab_v, ev_v, aux_v):
    wid = lax.axis_index("s") * NC + lax.axis_index("c")
    base = wid * ROWS

    pltpu.sync_copy(ev_hbm, ev_v)
    lane = lax.iota(jnp.int32, 16)
    zero = jnp.zeros((16,), jnp.int32)
    ev = ev_v[...]
    x_s = jnp.sum(jnp.where(lane == 0, ev, zero), dtype=jnp.int32)
    y_s = jnp.sum(jnp.where(lane == 1, ev, zero), dtype=jnp.int32)

    pltpu.sync_copy(mat_hbm.at[pl.ds(base, ROWS)], slab_v)
    own = (lane == 0) & (x_s >= base) & (x_s < base + ROWS)
    plsc.store_scatter(slab_v, [zero + (x_s - base), zero + y_s], zero, mask=own)
    pltpu.sync_copy(slab_v, out_hbm.at[pl.ds(base, ROWS)])

    @pl.when(wid == 0)
    def _():
        aux_v[...] = ev_v[...].astype(jnp.float32)
        pltpu.sync_copy(aux_v, aux_hbm)


def kernel(event, neighbour_matrix):
    ev16 = jnp.zeros((16,), jnp.int32).at[:4].set(event.astype(jnp.int32))
    new_matrix, aux = _graphgen_sc(ev16, neighbour_matrix)
    nodes = aux[:3].reshape(1, 3)
    features = aux[3:4].reshape(1, 1)
    edges = jnp.zeros((1, 2), jnp.int32)
    return nodes, features, edges, new_matrix


# speedup vs baseline: 1.1090x; 1.1090x over previous
import functools

import jax
import jax.numpy as jnp
from jax import lax
from jax.experimental import pallas as pl
from jax.experimental.pallas import tpu as pltpu
from jax.experimental.pallas import tpu_sc as plsc

D = 512
NC = 2
NS = 16
NW = NC * NS
ROWS = D // NW

_mesh = plsc.VectorSubcoreMesh(core_axis_name="c", subcore_axis_name="s")


@functools.partial(
    pl.kernel,
    mesh=_mesh,
    out_type=(
        jax.ShapeDtypeStruct((D, D), jnp.int32),
        jax.ShapeDtypeStruct((1, 3), jnp.float32),
        jax.ShapeDtypeStruct((1, 1), jnp.float32),
        jax.ShapeDtypeStruct((1, 2), jnp.int32),
    ),
    scratch_types=[
        pltpu.VMEM((ROWS, D), jnp.int32),
        pltpu.VMEM((16,), jnp.int32),
        pltpu.VMEM((16,), jnp.float32),
        pltpu.VMEM((16,), jnp.float32),
        pltpu.VMEM((16,), jnp.int32),
        pltpu.SemaphoreType.DMA,
        pltpu.SemaphoreType.DMA,
    ],
    compiler_params=pltpu.CompilerParams(needs_layout_passes=False),
)
def _graphgen_sc(ev_hbm, mat_hbm, out_hbm, nodes_hbm, feat_hbm, edges_hbm,
                 slab_v, ev_v, aux_v, feat_v, zed_v, sem_ev, sem_slab):
    wid = lax.axis_index("s") * NC + lax.axis_index("c")
    base = wid * ROWS

    cp_ev = pltpu.async_copy(ev_hbm, ev_v, sem_ev)
    cp_slab = pltpu.async_copy(mat_hbm.at[pl.ds(base, ROWS)], slab_v, sem_slab)

    cp_ev.wait()
    lane = lax.iota(jnp.int32, 16)
    zero = jnp.zeros((16,), jnp.int32)
    ev = ev_v[...]
    x_s = jnp.sum(jnp.where(lane == 0, ev, zero), dtype=jnp.int32)
    y_s = jnp.sum(jnp.where(lane == 1, ev, zero), dtype=jnp.int32)

    cp_slab.wait()
    own = (lane == 0) & (x_s >= base) & (x_s < base + ROWS)
    plsc.store_scatter(slab_v, [zero + (x_s - base), zero + y_s], zero, mask=own)
    pltpu.sync_copy(slab_v, out_hbm.at[pl.ds(base, ROWS)])

    @pl.when(wid == 0)
    def _():
        evf = ev.astype(jnp.float32)
        f_s = jnp.sum(jnp.where(lane == 3, evf, jnp.zeros((16,), jnp.float32)))
        aux_v[...] = evf
        feat_v[...] = jnp.zeros((16,), jnp.float32) + f_s
        zed_v[...] = zero
        i0 = jnp.int32(0)
        pltpu.sync_copy(aux_v.at[pl.ds(0, 3)], nodes_hbm.at[i0])
        pltpu.sync_copy(feat_v.at[pl.ds(0, 1)], feat_hbm.at[i0])
        pltpu.sync_copy(zed_v.at[pl.ds(0, 2)], edges_hbm.at[i0])


def kernel(event, neighbour_matrix):
    ev16 = jnp.zeros((16,), jnp.int32).at[:4].set(event.astype(jnp.int32))
    new_matrix, nodes, features, edges = _graphgen_sc(ev16, neighbour_matrix)
    return nodes, features, edges, new_matrix
